# baseline (speedup 1.0000x reference)
import functools

import jax
import jax.numpy as jnp
from jax import lax
from jax.experimental import pallas as pl
from jax.experimental.pallas import tpu as pltpu
from jax.experimental.pallas import tpu_sc as plsc

N = 10000
T = 3
E = 320000
D_IN = 128
D_H = 128
D_CH = 256
D_CLS = 128
SLOPE = (1.0 / 8.0 + 1.0 / 3.0) / 2.0

NC = 2
NS = 16
NW = NC * NS
EPW = E // NW
NPAD = 10240
RPS = NPAD // NS

E_PAD = 327680
EPW_P = E_PAD // NW
CCB = 160
NCHUNK_C = EPW_P // CCB
SCB = 160
NCHUNK_S = EPW_P // SCB
CH = 2000
NCHUNK_H = EPW // CH

_MESH = plsc.VectorSubcoreMesh(core_axis_name="c", subcore_axis_name="s")
f32 = jnp.float32
i32 = jnp.int32


def _dot(a, b):
    return lax.dot(a, b)


def _gru_body(w0, w1, p0, p1, out):
    for layer in range(2):
        p = p0 if layer == 0 else p1
        Wu, Uu, bu = p[0], p[1], p[2]
        Wr, Ur, br = p[3], p[4], p[5]
        Wh, Uh, bh = p[6], p[7], p[8]
        WUu = Wu + Uu
        WUr = Wr + Ur
        Q = w0[...] if layer == 0 else w1[...]
        for t in range(T):
            upd = jax.nn.sigmoid(_dot(WUu, Q) + bu)
            rst = jax.nn.sigmoid(_dot(WUr, Q) + br)
            hcap = jnp.tanh(_dot(Wh, Q) + _dot(Uh, rst * Q) + bh)
            Q = (1.0 - upd) * Q + upd * hcap
            out[layer * T + t] = Q


def _evolve_weights(gcn_W0, gcn_W1, p0, p1):
    return pl.pallas_call(
        _gru_body,
        out_shape=jax.ShapeDtypeStruct((2 * T, D_H, D_H), f32),
    )(gcn_W0, gcn_W1, p0, p1)


def _deg_kernel_body(s0, d0, s1, d1, s2, d2, out_hbm,
                     idx_v, ones_v, zero_v,
                     h0, h1, h2, h3, h4, h5):
    c = lax.axis_index("c")
    s = lax.axis_index("s")
    wid = s * NC + c
    hists = (h0, h1, h2, h3, h4, h5)
    idx_arrays = (s0, d0, s1, d1, s2, d2)

    @pl.loop(0, CH // 16)
    def _(i):
        ones_v[pl.ds(i * 16, 16)] = jnp.ones((16,), f32)

    @pl.loop(0, (NPAD // NS) // 16)
    def _(i):
        zero_v[pl.ds(i * 16, 16)] = jnp.zeros((16,), f32)

    for a in range(6):
        pltpu.sync_copy(zero_v, hists[a].at[pl.ds(s * (NPAD // NS), NPAD // NS)])
    plsc.subcore_barrier()

    for a in range(6):
        arr = idx_arrays[a]
        hist = hists[a]

        @pl.loop(0, NCHUNK_H)
        def _(k):
            off = wid * EPW + k * CH
            pltpu.sync_copy(arr.at[pl.ds(off, CH)], idx_v)
            pltpu.sync_copy(ones_v, hist.at[idx_v], add=True)

    plsc.subcore_barrier()

    span = NPAD // NS
    for a in range(6):
        pltpu.sync_copy(hists[a].at[pl.ds(s * span, span)],
                        out_hbm.at[c].at[a].at[pl.ds(s * span, span)])


def _degrees(s0, d0, s1, d1, s2, d2):
    k = pl.kernel(
        _deg_kernel_body,
        out_type=jax.ShapeDtypeStruct((NC, 6, NPAD), f32),
        mesh=_MESH,
        scratch_types=[
            pltpu.VMEM((CH,), i32),
            pltpu.VMEM((CH,), f32),
            pltpu.VMEM((NPAD // NS,), f32),
        ] + [pltpu.VMEM_SHARED((NPAD,), f32) for _ in range(6)],
    )
    return k(s0, d0, s1, d1, s2, d2)


def _combine_body(p_ref, out_ref):
    deg = jnp.maximum(p_ref[0] + p_ref[1], 1.0)
    out_ref[...] = lax.rsqrt(deg)


def _deg_scales(partials):
    return pl.pallas_call(
        _combine_body,
        out_shape=jax.ShapeDtypeStruct((6, NPAD), f32),
    )(partials)


def _conv_kernel_body(h_hbm, src_hbm, dst_hbm, out_hbm,
                      sidx0, sidx1, didx0, didx1, rows0, rows1,
                      acc_sh, gsem0, gsem1):
    c = lax.axis_index("c")
    s = lax.axis_index("s")
    wid = s * NC + c

    @pl.loop(0, CCB)
    def _(r):
        @pl.loop(0, D_H // 16)
        def _(cc):
            rows0.at[pl.ds(r, 1), pl.ds(cc * 16, 16)][...] = (
                jnp.zeros((1, 16), f32))

    @pl.loop(0, RPS // CCB)
    def _(b):
        pltpu.sync_copy(rows0, acc_sh.at[pl.ds(s * RPS + b * CCB, CCB)])

    plsc.subcore_barrier()

    base = wid * EPW_P
    sidx = (sidx0, sidx1)
    didx = (didx0, didx1)
    rows = (rows0, rows1)
    gsem = (gsem0, gsem1)

    def start(buf, k):
        off = base + k * CCB
        pltpu.sync_copy(src_hbm.at[pl.ds(off, CCB)], sidx[buf])
        pltpu.sync_copy(dst_hbm.at[pl.ds(off, CCB)], didx[buf])
        pltpu.make_async_copy(h_hbm.at[sidx[buf]], rows[buf],
                              gsem[buf]).start()

    def wait_g(buf):
        pltpu.make_async_copy(h_hbm.at[sidx[buf]], rows[buf],
                              gsem[buf]).wait()

    start(0, 0)

    @pl.loop(0, NCHUNK_C // 2)
    def _(i):
        k0 = 2 * i
        start(1, k0 + 1)
        wait_g(0)
        pltpu.sync_copy(rows[0], acc_sh.at[didx[0]], add=True)
        knext = jnp.minimum(k0 + 2, NCHUNK_C - 2)
        start(0, knext)
        wait_g(1)
        pltpu.sync_copy(rows[1], acc_sh.at[didx[1]], add=True)

    wait_g(0)
    plsc.subcore_barrier()

    pltpu.sync_copy(acc_sh.at[pl.ds(s * RPS, RPS)],
                    out_hbm.at[c].at[pl.ds(s * RPS, RPS)])


def _conv_aggregate(h, src, dst):
    k = pl.kernel(
        _conv_kernel_body,
        out_type=jax.ShapeDtypeStruct((NC, NPAD, D_H), f32),
        mesh=_MESH,
        scratch_types=[
            pltpu.VMEM((CCB,), i32),
            pltpu.VMEM((CCB,), i32),
            pltpu.VMEM((CCB,), i32),
            pltpu.VMEM((CCB,), i32),
            pltpu.VMEM((CCB, D_H), f32),
            pltpu.VMEM((CCB, D_H), f32),
            pltpu.VMEM_SHARED((NPAD, D_H), f32),
            pltpu.SemaphoreType.DMA,
            pltpu.SemaphoreType.DMA,
        ],
    )
    return k(h, src, dst)


def _score_dot_body(h_hbm, s0_hbm, d0_hbm, s1_hbm, d1_hbm, po_hbm, ne_hbm,
                    sidx0, sidx1, didx0, didx1,
                    rs0, rs1, rd0, rd1, pt0, pt1, gsem, hsem):
    c = lax.axis_index("c")
    s = lax.axis_index("s")
    wid = s * NC + c
    base = wid * EPW_P
    sidx = (sidx0, sidx1)
    didx = (didx0, didx1)
    rs = (rs0, rs1)
    rd = (rd0, rd1)
    pt = (pt0, pt1)

    for src_hbm, dst_hbm, out_hbm in ((s0_hbm, d0_hbm, po_hbm),
                                      (s1_hbm, d1_hbm, ne_hbm)):
        def start(buf, k):
            off = base + k * SCB
            pltpu.sync_copy(src_hbm.at[pl.ds(off, SCB)], sidx[buf])
            pltpu.sync_copy(dst_hbm.at[pl.ds(off, SCB)], didx[buf])
            pltpu.make_async_copy(h_hbm.at[sidx[buf]], rs[buf],
                                  gsem.at[buf]).start()
            pltpu.make_async_copy(h_hbm.at[didx[buf]], rd[buf],
                                  hsem.at[buf]).start()

        def wait_g(buf):
            pltpu.make_async_copy(h_hbm.at[sidx[buf]], rs[buf],
                                  gsem.at[buf]).wait()
            pltpu.make_async_copy(h_hbm.at[didx[buf]], rd[buf],
                                  hsem.at[buf]).wait()

        def dots(buf, k):
            a = rs[buf]
            b = rd[buf]
            p = pt[buf]

            @pl.loop(0, SCB, step=4)
            def _(r0):
                for u in range(4):
                    r = r0 + u
                    acc = (a.at[pl.ds(r, 1), pl.ds(0, 16)][...] *
                           b.at[pl.ds(r, 1), pl.ds(0, 16)][...])
                    for v in range(1, D_CLS // 16):
                        acc += (a.at[pl.ds(r, 1), pl.ds(v * 16, 16)][...] *
                                b.at[pl.ds(r, 1), pl.ds(v * 16, 16)][...])
                    p.at[pl.ds(r, 1), :][...] = acc

            off = base + k * SCB
            pltpu.sync_copy(p, out_hbm.at[pl.ds(off, SCB)])

        start(0, 0)

        @pl.loop(0, NCHUNK_S // 2)
        def _(i):
            k0 = 2 * i
            start(1, k0 + 1)
            wait_g(0)
            dots(0, k0)
            knext = jnp.minimum(k0 + 2, NCHUNK_S - 2)
            start(0, knext)
            wait_g(1)
            dots(1, k0 + 1)

        wait_g(0)


def _score_dots(h, s0, d0, s1, d1):
    out = jax.ShapeDtypeStruct((E_PAD, 16), f32)
    k = pl.kernel(
        _score_dot_body,
        out_type=(out, out),
        mesh=_MESH,
        scratch_types=[
            pltpu.VMEM((SCB,), i32),
            pltpu.VMEM((SCB,), i32),
            pltpu.VMEM((SCB,), i32),
            pltpu.VMEM((SCB,), i32),
            pltpu.VMEM((SCB, D_CLS), f32),
            pltpu.VMEM((SCB, D_CLS), f32),
            pltpu.VMEM((SCB, D_CLS), f32),
            pltpu.VMEM((SCB, D_CLS), f32),
            pltpu.VMEM((SCB, 16), f32),
            pltpu.VMEM((SCB, 16), f32),
            pltpu.SemaphoreType.DMA((2,)),
            pltpu.SemaphoreType.DMA((2,)),
        ],
    )
    return k(h, s0, d0, s1, d1)


_BM = 1024


def _mm1_body(x_ref, so_ref, w_ref, out_ref):
    out_ref[...] = _dot(x_ref[...] * so_ref[...], w_ref[...])


def _prematmul(x, so, w):
    return pl.pallas_call(
        _mm1_body,
        grid=(NPAD // _BM,),
        in_specs=[
            pl.BlockSpec((_BM, D_H), lambda i: (i, 0)),
            pl.BlockSpec((_BM, 1), lambda i: (i, 0)),
            pl.BlockSpec((D_H, D_H), lambda i: (0, 0)),
        ],
        out_specs=pl.BlockSpec((_BM, D_H), lambda i: (i, 0)),
        out_shape=jax.ShapeDtypeStruct((NPAD, D_H), f32),
    )(x, so, w)


def _epi_mm_body(p_ref, si_ref, b_ref, so_ref, w_ref, out_ref):
    agg = (p_ref[0] + p_ref[1]) * si_ref[...] + b_ref[...]
    x = jnp.where(agg >= 0, agg, SLOPE * agg)
    out_ref[...] = _dot(x * so_ref[...], w_ref[...])


def _epilogue_prematmul(partial, si, b, so, w):
    return pl.pallas_call(
        _epi_mm_body,
        grid=(NPAD // _BM,),
        in_specs=[
            pl.BlockSpec((NC, _BM, D_H), lambda i: (0, i, 0)),
            pl.BlockSpec((_BM, 1), lambda i: (i, 0)),
            pl.BlockSpec((1, D_H), lambda i: (0, 0)),
            pl.BlockSpec((_BM, 1), lambda i: (i, 0)),
            pl.BlockSpec((D_H, D_H), lambda i: (0, 0)),
        ],
        out_specs=pl.BlockSpec((_BM, D_H), lambda i: (i, 0)),
        out_shape=jax.ShapeDtypeStruct((NPAD, D_H), f32),
    )(partial, si, b, so, w)


def _epi_mlp_body(p_ref, si_ref, b_ref, w1_ref, b1_ref, w2_ref, b2_ref,
                  out_ref):
    agg = (p_ref[0] + p_ref[1]) * si_ref[...] + b_ref[...]
    x = jnp.where(agg >= 0, agg, SLOPE * agg)
    mid = jnp.maximum(_dot(x, w1_ref[...]) + b1_ref[...], 0.0)
    out_ref[...] = _dot(mid, w2_ref[...]) + b2_ref[...]


def _epilogue_mlp(partial, si, b, w1, b1, w2, b2):
    return pl.pallas_call(
        _epi_mlp_body,
        grid=(NPAD // _BM,),
        in_specs=[
            pl.BlockSpec((NC, _BM, D_H), lambda i: (0, i, 0)),
            pl.BlockSpec((_BM, 1), lambda i: (i, 0)),
            pl.BlockSpec((1, D_H), lambda i: (0, 0)),
            pl.BlockSpec((D_H, D_CH), lambda i: (0, 0)),
            pl.BlockSpec((1, D_CH), lambda i: (0, 0)),
            pl.BlockSpec((D_CH, D_CLS), lambda i: (0, 0)),
            pl.BlockSpec((1, D_CLS), lambda i: (0, 0)),
        ],
        out_specs=pl.BlockSpec((_BM, D_CLS), lambda i: (i, 0)),
        out_shape=jax.ShapeDtypeStruct((NPAD, D_CLS), f32),
    )(partial, si, b, w1, b1, w2, b2)


_BE = 4000


def _dots_body(a_ref, b_ref, pos_ref, neg_ref):
    pos_ref[...] = jnp.sum(a_ref[...], axis=1, keepdims=True)
    neg_ref[...] = jnp.sum(b_ref[...], axis=1, keepdims=True)


def _edge_dots(pp, np_):
    spec = pl.BlockSpec((_BE, 16), lambda i: (i, 0))
    ospec = pl.BlockSpec((_BE, 1), lambda i: (i, 0))
    return pl.pallas_call(
        _dots_body,
        grid=(E // _BE,),
        in_specs=[spec, spec],
        out_specs=(ospec, ospec),
        out_shape=(jax.ShapeDtypeStruct((E, 1), f32),
                   jax.ShapeDtypeStruct((E, 1), f32)),
    )(pp, np_)


def kernel(node_feature, edge_index, neg_edge_index, gcn_W0, gcn_W1,
           g0_Wu, g0_Uu, g0_bu, g0_Wr, g0_Ur, g0_br, g0_Wh, g0_Uh, g0_bh,
           g1_Wu, g1_Uu, g1_bu, g1_Wr, g1_Ur, g1_br, g1_Wh, g1_Uh, g1_bh,
           conv_b0, conv_b1, mlp_W1, mlp_b1, mlp_W2, mlp_b2):
    p0 = jnp.stack([g0_Wu, g0_Uu, g0_bu, g0_Wr, g0_Ur, g0_br,
                    g0_Wh, g0_Uh, g0_bh])
    p1 = jnp.stack([g1_Wu, g1_Uu, g1_bu, g1_Wr, g1_Ur, g1_br,
                    g1_Wh, g1_Uh, g1_bh])
    Ws = _evolve_weights(gcn_W0, gcn_W1, p0, p1)

    src = [edge_index[t, 0] for t in range(T)]
    dst = [edge_index[t, 1] for t in range(T)]
    nsrc = [neg_edge_index[t, 0] for t in range(T)]
    ndst = [neg_edge_index[t, 1] for t in range(T)]

    padn = E_PAD - E
    pad_gather = (jnp.arange(padn, dtype=i32) * 13) % N
    pad_dump = N + (jnp.arange(padn, dtype=i32) % (NPAD - N))
    srcp = [jnp.concatenate([a, pad_gather]) for a in src]
    dstp = [jnp.concatenate([a, pad_dump]) for a in dst]
    nsrcp = [jnp.concatenate([a, pad_gather]) for a in nsrc]
    ndstp = [jnp.concatenate([a, pad_gather]) for a in ndst]
    srcg = [jnp.concatenate([a, pad_gather]) for a in src]
    dstg = [jnp.concatenate([a, pad_gather]) for a in dst]

    partial_hists = _degrees(src[0], dst[0], src[1], dst[1], src[2], dst[2])
    scales = _deg_scales(partial_hists)
    so = [scales[2 * t].reshape(NPAD, 1) for t in range(T)]
    si = [scales[2 * t + 1].reshape(NPAD, 1) for t in range(T)]

    b0 = conv_b0.reshape(1, D_H)
    b1 = conv_b1.reshape(1, D_H)
    mb1 = mlp_b1.reshape(1, D_CH)
    mb2 = mlp_b2.reshape(1, D_CLS)

    pos = []
    neg = []
    for t in range(T):
        xp = jnp.pad(node_feature[t], ((0, NPAD - N), (0, 0)))
        h = _prematmul(xp, so[t], Ws[t])
        part = _conv_aggregate(h, srcp[t], dstp[t])
        h2 = _epilogue_prematmul(part, si[t], b0, so[t], Ws[T + t])
        part2 = _conv_aggregate(h2, srcp[t], dstp[t])
        H = _epilogue_mlp(part2, si[t], b1, mlp_W1, mb1, mlp_W2, mb2)
        pp, np_ = _score_dots(H, srcg[t], dstg[t], nsrcp[t], ndstp[t])
        p, n = _edge_dots(pp, np_)
        pos.append(p)
        neg.append(n)

    return (jnp.concatenate(pos, 0), jnp.concatenate(neg, 0))


# speedup vs baseline: 1.0564x; 1.0564x over previous
import functools

import jax
import jax.numpy as jnp
from jax import lax
from jax.experimental import pallas as pl
from jax.experimental.pallas import tpu as pltpu
from jax.experimental.pallas import tpu_sc as plsc

N = 10000
T = 3
E = 320000
D_IN = 128
D_H = 128
D_CH = 256
D_CLS = 128
SLOPE = (1.0 / 8.0 + 1.0 / 3.0) / 2.0

NC = 2
NS = 16
NW = NC * NS
EPW = E // NW
NPAD = 10240
RPS = NPAD // NS

E_PAD = 327680
EPW_P = E_PAD // NW
CCB = 160
NCHUNK_C = EPW_P // CCB
SCB = 160
NCHUNK_S = EPW_P // SCB
CH = 2000
NCHUNK_H = EPW // CH

_MESH = plsc.VectorSubcoreMesh(core_axis_name="c", subcore_axis_name="s")
f32 = jnp.float32
i32 = jnp.int32


def _dot(a, b):
    return lax.dot(a, b)


def _gru_body(w0, w1, p0, p1, out):
    for layer in range(2):
        p = p0 if layer == 0 else p1
        Wu, Uu, bu = p[0], p[1], p[2]
        Wr, Ur, br = p[3], p[4], p[5]
        Wh, Uh, bh = p[6], p[7], p[8]
        WUu = Wu + Uu
        WUr = Wr + Ur
        Q = w0[...] if layer == 0 else w1[...]
        for t in range(T):
            upd = jax.nn.sigmoid(_dot(WUu, Q) + bu)
            rst = jax.nn.sigmoid(_dot(WUr, Q) + br)
            hcap = jnp.tanh(_dot(Wh, Q) + _dot(Uh, rst * Q) + bh)
            Q = (1.0 - upd) * Q + upd * hcap
            out[layer * T + t] = Q


def _evolve_weights(gcn_W0, gcn_W1, p0, p1):
    return pl.pallas_call(
        _gru_body,
        out_shape=jax.ShapeDtypeStruct((2 * T, D_H, D_H), f32),
    )(gcn_W0, gcn_W1, p0, p1)


def _deg_kernel_body(s0, d0, s1, d1, s2, d2, out_hbm,
                     idx_v, ones_v, zero_v,
                     h0, h1, h2, h3, h4, h5):
    c = lax.axis_index("c")
    s = lax.axis_index("s")
    wid = s * NC + c
    hists = (h0, h1, h2, h3, h4, h5)
    idx_arrays = (s0, d0, s1, d1, s2, d2)

    @pl.loop(0, CH // 16)
    def _(i):
        ones_v[pl.ds(i * 16, 16)] = jnp.ones((16,), f32)

    @pl.loop(0, (NPAD // NS) // 16)
    def _(i):
        zero_v[pl.ds(i * 16, 16)] = jnp.zeros((16,), f32)

    for a in range(6):
        pltpu.sync_copy(zero_v, hists[a].at[pl.ds(s * (NPAD // NS), NPAD // NS)])
    plsc.subcore_barrier()

    for a in range(6):
        arr = idx_arrays[a]
        hist = hists[a]

        @pl.loop(0, NCHUNK_H)
        def _(k):
            off = wid * EPW + k * CH
            pltpu.sync_copy(arr.at[pl.ds(off, CH)], idx_v)
            pltpu.sync_copy(ones_v, hist.at[idx_v], add=True)

    plsc.subcore_barrier()

    span = NPAD // NS
    for a in range(6):
        pltpu.sync_copy(hists[a].at[pl.ds(s * span, span)],
                        out_hbm.at[c].at[a].at[pl.ds(s * span, span)])


def _degrees(s0, d0, s1, d1, s2, d2):
    k = pl.kernel(
        _deg_kernel_body,
        out_type=jax.ShapeDtypeStruct((NC, 6, NPAD), f32),
        mesh=_MESH,
        scratch_types=[
            pltpu.VMEM((CH,), i32),
            pltpu.VMEM((CH,), f32),
            pltpu.VMEM((NPAD // NS,), f32),
        ] + [pltpu.VMEM_SHARED((NPAD,), f32) for _ in range(6)],
    )
    return k(s0, d0, s1, d1, s2, d2)


def _combine_body(p_ref, out_ref):
    deg = jnp.maximum(p_ref[0] + p_ref[1], 1.0)
    out_ref[...] = lax.rsqrt(deg)


def _deg_scales(partials):
    return pl.pallas_call(
        _combine_body,
        out_shape=jax.ShapeDtypeStruct((6, NPAD), f32),
    )(partials)


def _conv_kernel_body(h_hbm, src_hbm, dst_hbm, out_hbm,
                      sidx0, sidx1, didx0, didx1, rows0, rows1,
                      acc_sh, gsem0, gsem1, isem0, isem1):
    c = lax.axis_index("c")
    s = lax.axis_index("s")
    wid = s * NC + c

    @pl.loop(0, CCB)
    def _(r):
        @pl.loop(0, D_H // 16)
        def _(cc):
            rows0.at[pl.ds(r, 1), pl.ds(cc * 16, 16)][...] = (
                jnp.zeros((1, 16), f32))

    @pl.loop(0, RPS // CCB)
    def _(b):
        pltpu.sync_copy(rows0, acc_sh.at[pl.ds(s * RPS + b * CCB, CCB)])

    plsc.subcore_barrier()

    base = wid * EPW_P
    sidx = (sidx0, sidx1)
    didx = (didx0, didx1)
    rows = (rows0, rows1)
    gsem = (gsem0, gsem1)

    isem = (isem0, isem1)

    def prefetch(buf, k):
        off = base + k * CCB
        pltpu.make_async_copy(src_hbm.at[pl.ds(off, CCB)], sidx[buf],
                              isem[buf]).start()
        pltpu.make_async_copy(dst_hbm.at[pl.ds(off, CCB)], didx[buf],
                              isem[buf]).start()

    def launchg(buf, k):
        off = base + k * CCB
        pltpu.make_async_copy(src_hbm.at[pl.ds(off, CCB)], sidx[buf],
                              isem[buf]).wait()
        pltpu.make_async_copy(dst_hbm.at[pl.ds(off, CCB)], didx[buf],
                              isem[buf]).wait()
        pltpu.make_async_copy(h_hbm.at[sidx[buf]], rows[buf],
                              gsem[buf]).start()

    def wait_g(buf):
        pltpu.make_async_copy(h_hbm.at[sidx[buf]], rows[buf],
                              gsem[buf]).wait()

    prefetch(0, 0)
    launchg(0, 0)

    @pl.loop(0, NCHUNK_C // 2)
    def _(i):
        k0 = 2 * i
        prefetch(1, k0 + 1)
        wait_g(0)
        launchg(1, k0 + 1)
        pltpu.sync_copy(rows[0], acc_sh.at[didx[0]], add=True)
        knext = jnp.minimum(k0 + 2, NCHUNK_C - 2)
        prefetch(0, knext)
        wait_g(1)
        launchg(0, knext)
        pltpu.sync_copy(rows[1], acc_sh.at[didx[1]], add=True)

    wait_g(0)
    plsc.subcore_barrier()

    pltpu.sync_copy(acc_sh.at[pl.ds(s * RPS, RPS)],
                    out_hbm.at[c].at[pl.ds(s * RPS, RPS)])


def _conv_aggregate(h, src, dst):
    k = pl.kernel(
        _conv_kernel_body,
        out_type=jax.ShapeDtypeStruct((NC, NPAD, D_H), f32),
        mesh=_MESH,
        scratch_types=[
            pltpu.VMEM((CCB,), i32),
            pltpu.VMEM((CCB,), i32),
            pltpu.VMEM((CCB,), i32),
            pltpu.VMEM((CCB,), i32),
            pltpu.VMEM((CCB, D_H), f32),
            pltpu.VMEM((CCB, D_H), f32),
            pltpu.VMEM_SHARED((NPAD, D_H), f32),
            pltpu.SemaphoreType.DMA,
            pltpu.SemaphoreType.DMA,
            pltpu.SemaphoreType.DMA,
            pltpu.SemaphoreType.DMA,
        ],
    )
    return k(h, src, dst)


def _score_dot_body(h_hbm, s0_hbm, d0_hbm, s1_hbm, d1_hbm, po_hbm, ne_hbm,
                    sidx0, sidx1, didx0, didx1,
                    rs0, rs1, rd0, rd1, pt0, pt1, gsem, hsem, isem, wsem):
    c = lax.axis_index("c")
    s = lax.axis_index("s")
    wid = s * NC + c
    base = wid * EPW_P
    sidx = (sidx0, sidx1)
    didx = (didx0, didx1)
    rs = (rs0, rs1)
    rd = (rd0, rd1)
    pt = (pt0, pt1)

    for src_hbm, dst_hbm, out_hbm in ((s0_hbm, d0_hbm, po_hbm),
                                      (s1_hbm, d1_hbm, ne_hbm)):
        def prefetch(buf, k):
            off = base + k * SCB
            pltpu.make_async_copy(src_hbm.at[pl.ds(off, SCB)], sidx[buf],
                                  isem.at[buf]).start()
            pltpu.make_async_copy(dst_hbm.at[pl.ds(off, SCB)], didx[buf],
                                  isem.at[buf]).start()

        def launchg(buf, k):
            off = base + k * SCB
            pltpu.make_async_copy(src_hbm.at[pl.ds(off, SCB)], sidx[buf],
                                  isem.at[buf]).wait()
            pltpu.make_async_copy(dst_hbm.at[pl.ds(off, SCB)], didx[buf],
                                  isem.at[buf]).wait()
            pltpu.make_async_copy(h_hbm.at[sidx[buf]], rs[buf],
                                  gsem.at[buf]).start()
            pltpu.make_async_copy(h_hbm.at[didx[buf]], rd[buf],
                                  hsem.at[buf]).start()

        def wait_g(buf):
            pltpu.make_async_copy(h_hbm.at[sidx[buf]], rs[buf],
                                  gsem.at[buf]).wait()
            pltpu.make_async_copy(h_hbm.at[didx[buf]], rd[buf],
                                  hsem.at[buf]).wait()

        def write_part(buf, k):
            off = base + k * SCB
            pltpu.make_async_copy(pt[buf], out_hbm.at[pl.ds(off, SCB)],
                                  wsem.at[buf]).start()

        def wait_w(buf, k):
            off = base + k * SCB
            pltpu.make_async_copy(pt[buf], out_hbm.at[pl.ds(off, SCB)],
                                  wsem.at[buf]).wait()

        def dots(buf, k):
            a = rs[buf]
            b = rd[buf]
            p = pt[buf]
            wait_w(buf, k)

            @pl.loop(0, SCB, step=4)
            def _(r0):
                for u in range(4):
                    r = r0 + u
                    acc = (a.at[pl.ds(r, 1), pl.ds(0, 16)][...] *
                           b.at[pl.ds(r, 1), pl.ds(0, 16)][...])
                    for v in range(1, D_CLS // 16):
                        acc += (a.at[pl.ds(r, 1), pl.ds(v * 16, 16)][...] *
                                b.at[pl.ds(r, 1), pl.ds(v * 16, 16)][...])
                    p.at[pl.ds(r, 1), :][...] = acc

            write_part(buf, k)

        prefetch(0, 0)
        launchg(0, 0)
        write_part(0, 0)
        write_part(1, 1)

        @pl.loop(0, NCHUNK_S // 2)
        def _(i):
            k0 = 2 * i
            prefetch(1, k0 + 1)
            wait_g(0)
            launchg(1, k0 + 1)
            dots(0, k0)
            knext = jnp.minimum(k0 + 2, NCHUNK_S - 2)
            prefetch(0, knext)
            wait_g(1)
            launchg(0, knext)
            dots(1, k0 + 1)

        wait_g(0)
        wait_w(0, NCHUNK_S - 2)
        wait_w(1, NCHUNK_S - 1)


def _score_dots(h, s0, d0, s1, d1):
    out = jax.ShapeDtypeStruct((E_PAD, 16), f32)
    k = pl.kernel(
        _score_dot_body,
        out_type=(out, out),
        mesh=_MESH,
        scratch_types=[
            pltpu.VMEM((SCB,), i32),
            pltpu.VMEM((SCB,), i32),
            pltpu.VMEM((SCB,), i32),
            pltpu.VMEM((SCB,), i32),
            pltpu.VMEM((SCB, D_CLS), f32),
            pltpu.VMEM((SCB, D_CLS), f32),
            pltpu.VMEM((SCB, D_CLS), f32),
            pltpu.VMEM((SCB, D_CLS), f32),
            pltpu.VMEM((SCB, 16), f32),
            pltpu.VMEM((SCB, 16), f32),
            pltpu.SemaphoreType.DMA((2,)),
            pltpu.SemaphoreType.DMA((2,)),
            pltpu.SemaphoreType.DMA((2,)),
            pltpu.SemaphoreType.DMA((2,)),
        ],
    )
    return k(h, s0, d0, s1, d1)


_BM = 1024


def _mm1_body(x_ref, so_ref, w_ref, out_ref):
    out_ref[...] = _dot(x_ref[...] * so_ref[...], w_ref[...])


def _prematmul(x, so, w):
    return pl.pallas_call(
        _mm1_body,
        grid=(NPAD // _BM,),
        in_specs=[
            pl.BlockSpec((_BM, D_H), lambda i: (i, 0)),
            pl.BlockSpec((_BM, 1), lambda i: (i, 0)),
            pl.BlockSpec((D_H, D_H), lambda i: (0, 0)),
        ],
        out_specs=pl.BlockSpec((_BM, D_H), lambda i: (i, 0)),
        out_shape=jax.ShapeDtypeStruct((NPAD, D_H), f32),
    )(x, so, w)


def _epi_mm_body(p_ref, si_ref, b_ref, so_ref, w_ref, out_ref):
    agg = (p_ref[0] + p_ref[1]) * si_ref[...] + b_ref[...]
    x = jnp.where(agg >= 0, agg, SLOPE * agg)
    out_ref[...] = _dot(x * so_ref[...], w_ref[...])


def _epilogue_prematmul(partial, si, b, so, w):
    return pl.pallas_call(
        _epi_mm_body,
        grid=(NPAD // _BM,),
        in_specs=[
            pl.BlockSpec((NC, _BM, D_H), lambda i: (0, i, 0)),
            pl.BlockSpec((_BM, 1), lambda i: (i, 0)),
            pl.BlockSpec((1, D_H), lambda i: (0, 0)),
            pl.BlockSpec((_BM, 1), lambda i: (i, 0)),
            pl.BlockSpec((D_H, D_H), lambda i: (0, 0)),
        ],
        out_specs=pl.BlockSpec((_BM, D_H), lambda i: (i, 0)),
        out_shape=jax.ShapeDtypeStruct((NPAD, D_H), f32),
    )(partial, si, b, so, w)


def _epi_mlp_body(p_ref, si_ref, b_ref, w1_ref, b1_ref, w2_ref, b2_ref,
                  out_ref):
    agg = (p_ref[0] + p_ref[1]) * si_ref[...] + b_ref[...]
    x = jnp.where(agg >= 0, agg, SLOPE * agg)
    mid = jnp.maximum(_dot(x, w1_ref[...]) + b1_ref[...], 0.0)
    out_ref[...] = _dot(mid, w2_ref[...]) + b2_ref[...]


def _epilogue_mlp(partial, si, b, w1, b1, w2, b2):
    return pl.pallas_call(
        _epi_mlp_body,
        grid=(NPAD // _BM,),
        in_specs=[
            pl.BlockSpec((NC, _BM, D_H), lambda i: (0, i, 0)),
            pl.BlockSpec((_BM, 1), lambda i: (i, 0)),
            pl.BlockSpec((1, D_H), lambda i: (0, 0)),
            pl.BlockSpec((D_H, D_CH), lambda i: (0, 0)),
            pl.BlockSpec((1, D_CH), lambda i: (0, 0)),
            pl.BlockSpec((D_CH, D_CLS), lambda i: (0, 0)),
            pl.BlockSpec((1, D_CLS), lambda i: (0, 0)),
        ],
        out_specs=pl.BlockSpec((_BM, D_CLS), lambda i: (i, 0)),
        out_shape=jax.ShapeDtypeStruct((NPAD, D_CLS), f32),
    )(partial, si, b, w1, b1, w2, b2)


_BE = 4000


def _dots_body(a_ref, b_ref, pos_ref, neg_ref):
    pos_ref[...] = jnp.sum(a_ref[...], axis=1, keepdims=True)
    neg_ref[...] = jnp.sum(b_ref[...], axis=1, keepdims=True)


def _edge_dots(pp, np_):
    spec = pl.BlockSpec((_BE, 16), lambda i: (i, 0))
    ospec = pl.BlockSpec((_BE, 1), lambda i: (i, 0))
    return pl.pallas_call(
        _dots_body,
        grid=(E // _BE,),
        in_specs=[spec, spec],
        out_specs=(ospec, ospec),
        out_shape=(jax.ShapeDtypeStruct((E, 1), f32),
                   jax.ShapeDtypeStruct((E, 1), f32)),
    )(pp, np_)


def kernel(node_feature, edge_index, neg_edge_index, gcn_W0, gcn_W1,
           g0_Wu, g0_Uu, g0_bu, g0_Wr, g0_Ur, g0_br, g0_Wh, g0_Uh, g0_bh,
           g1_Wu, g1_Uu, g1_bu, g1_Wr, g1_Ur, g1_br, g1_Wh, g1_Uh, g1_bh,
           conv_b0, conv_b1, mlp_W1, mlp_b1, mlp_W2, mlp_b2):
    p0 = jnp.stack([g0_Wu, g0_Uu, g0_bu, g0_Wr, g0_Ur, g0_br,
                    g0_Wh, g0_Uh, g0_bh])
    p1 = jnp.stack([g1_Wu, g1_Uu, g1_bu, g1_Wr, g1_Ur, g1_br,
                    g1_Wh, g1_Uh, g1_bh])
    Ws = _evolve_weights(gcn_W0, gcn_W1, p0, p1)

    src = [edge_index[t, 0] for t in range(T)]
    dst = [edge_index[t, 1] for t in range(T)]
    nsrc = [neg_edge_index[t, 0] for t in range(T)]
    ndst = [neg_edge_index[t, 1] for t in range(T)]

    padn = E_PAD - E
    pad_gather = (jnp.arange(padn, dtype=i32) * 13) % N
    pad_dump = N + (jnp.arange(padn, dtype=i32) % (NPAD - N))
    srcp = [jnp.concatenate([a, pad_gather]) for a in src]
    dstp = [jnp.concatenate([a, pad_dump]) for a in dst]
    nsrcp = [jnp.concatenate([a, pad_gather]) for a in nsrc]
    ndstp = [jnp.concatenate([a, pad_gather]) for a in ndst]
    srcg = [jnp.concatenate([a, pad_gather]) for a in src]
    dstg = [jnp.concatenate([a, pad_gather]) for a in dst]

    partial_hists = _degrees(src[0], dst[0], src[1], dst[1], src[2], dst[2])
    scales = _deg_scales(partial_hists)
    so = [scales[2 * t].reshape(NPAD, 1) for t in range(T)]
    si = [scales[2 * t + 1].reshape(NPAD, 1) for t in range(T)]

    b0 = conv_b0.reshape(1, D_H)
    b1 = conv_b1.reshape(1, D_H)
    mb1 = mlp_b1.reshape(1, D_CH)
    mb2 = mlp_b2.reshape(1, D_CLS)

    pos = []
    neg = []
    for t in range(T):
        xp = jnp.pad(node_feature[t], ((0, NPAD - N), (0, 0)))
        h = _prematmul(xp, so[t], Ws[t])
        part = _conv_aggregate(h, srcp[t], dstp[t])
        h2 = _epilogue_prematmul(part, si[t], b0, so[t], Ws[T + t])
        part2 = _conv_aggregate(h2, srcp[t], dstp[t])
        H = _epilogue_mlp(part2, si[t], b1, mlp_W1, mb1, mlp_W2, mb2)
        pp, np_ = _score_dots(H, srcg[t], dstg[t], nsrcp[t], ndstp[t])
        p, n = _edge_dots(pp, np_)
        pos.append(p)
        neg.append(n)

    return (jnp.concatenate(pos, 0), jnp.concatenate(neg, 0))
